# baseline (speedup 1.0000x reference)
import functools

import jax
import jax.numpy as jnp
from jax import lax
from jax.experimental import pallas as pl
from jax.experimental.pallas import tpu as pltpu
from jax.experimental.pallas import tpu_sc as plsc

N = 10000
E = 320000
D_IN = 128
D_H = 128
D_OUT = 64

NC = 2
NS = 16
NTILES = NC * NS
NP = 10240
EP = NP * NTILES
CHUNK = 128
EPA = EP // NTILES
NCHA = EPA // CHUNK
EPD = EP // NS
NCHD = EPD // CHUNK
RPT = NP // NS
WB = 128
WBN = RPT // WB

_MESH = plsc.VectorSubcoreMesh(
    core_axis_name="c", subcore_axis_name="s", num_cores=NC, num_subcores=NS
)

BN = 1024
GRID = NP // BN


@functools.partial(
    pl.kernel,
    out_type=jax.ShapeDtypeStruct((NC, NP, D_H), jnp.float32),
    mesh=_MESH,
    scratch_types=[
        pltpu.VMEM((CHUNK,), jnp.int32),
        pltpu.VMEM((CHUNK, D_H), jnp.float32),
        pltpu.VMEM((WB, D_H), jnp.float32),
        pltpu.VMEM_SHARED((NP, D_H), jnp.float32),
    ],
)
def _degree_kernel(ef_hbm, out_hbm, sidx, ones_v, zb, acc_sh):
    c = lax.axis_index("c")
    s = lax.axis_index("s")
    zero = jnp.zeros((16,), jnp.float32)
    one = jnp.ones((16,), jnp.float32)

    @pl.loop(0, WB)
    def _(i):
        for j in range(D_H // 16):
            zb[i, pl.ds(j * 16, 16)] = zero

    @pl.loop(0, CHUNK)
    def _(i):
        for j in range(D_H // 16):
            ones_v[i, pl.ds(j * 16, 16)] = one

    r0 = s * RPT
    for k in range(WBN):
        pltpu.sync_copy(zb, acc_sh.at[pl.ds(r0 + k * WB, WB)])
    plsc.subcore_barrier()

    base = c * EP + s * EPD

    @pl.loop(0, NCHD)
    def _(ch):
        off = base + ch * CHUNK
        pltpu.sync_copy(ef_hbm.at[pl.ds(off, CHUNK)], sidx)
        pltpu.sync_copy(ones_v, acc_sh.at[sidx], add=True)

    plsc.subcore_barrier()
    for k in range(WBN):
        pltpu.sync_copy(acc_sh.at[pl.ds(r0 + k * WB, WB)], zb)
        pltpu.sync_copy(zb, out_hbm.at[c, pl.ds(r0 + k * WB, WB)])


@functools.partial(
    pl.kernel,
    out_type=jax.ShapeDtypeStruct((NC, NP, D_H), jnp.float32),
    mesh=_MESH,
    scratch_types=[
        pltpu.VMEM((CHUNK,), jnp.int32),
        pltpu.VMEM((CHUNK,), jnp.int32),
        pltpu.VMEM((CHUNK,), jnp.int32),
        pltpu.VMEM((CHUNK,), jnp.int32),
        pltpu.VMEM((CHUNK, D_H), jnp.float32),
        pltpu.VMEM((CHUNK, D_H), jnp.float32),
        pltpu.VMEM_SHARED((NP, D_H), jnp.float32),
        pltpu.SemaphoreType.DMA,
        pltpu.SemaphoreType.DMA,
    ],
)
def _agg(g_hbm, src_hbm, dst_hbm, out_hbm, sidx0, didx0, sidx1, didx1,
         rows0, rows1, acc_sh, sem0, sem1):
    c = lax.axis_index("c")
    s = lax.axis_index("s")
    tid = s * NC + c
    zero = jnp.zeros((16,), jnp.float32)

    @pl.loop(0, WB)
    def _(i):
        for j in range(D_H // 16):
            rows0[i, pl.ds(j * 16, 16)] = zero

    r0 = s * RPT
    for k in range(WBN):
        pltpu.sync_copy(rows0, acc_sh.at[pl.ds(r0 + k * WB, WB)])
    plsc.subcore_barrier()

    base = tid * EPA

    pltpu.sync_copy(src_hbm.at[pl.ds(base, CHUNK)], sidx0)
    pltpu.sync_copy(dst_hbm.at[pl.ds(base, CHUNK)], didx0)
    pltpu.async_copy(g_hbm.at[sidx0], rows0, sem0)

    nhalf = NCHA // 2

    @pl.loop(0, nhalf)
    def _(i):
        ch = i * 2
        off1 = base + (ch + 1) * CHUNK
        pltpu.sync_copy(src_hbm.at[pl.ds(off1, CHUNK)], sidx1)
        pltpu.sync_copy(dst_hbm.at[pl.ds(off1, CHUNK)], didx1)
        pltpu.make_async_copy(g_hbm.at[sidx0], rows0, sem0).wait()
        pltpu.async_copy(g_hbm.at[sidx1], rows1, sem1)
        pltpu.sync_copy(rows0, acc_sh.at[didx0], add=True)

        @pl.when(i < nhalf - 1)
        def _():
            off2 = base + (ch + 2) * CHUNK
            pltpu.sync_copy(src_hbm.at[pl.ds(off2, CHUNK)], sidx0)
            pltpu.sync_copy(dst_hbm.at[pl.ds(off2, CHUNK)], didx0)

        pltpu.make_async_copy(g_hbm.at[sidx1], rows1, sem1).wait()

        @pl.when(i < nhalf - 1)
        def _():
            pltpu.async_copy(g_hbm.at[sidx0], rows0, sem0)

        pltpu.sync_copy(rows1, acc_sh.at[didx1], add=True)

    plsc.subcore_barrier()
    for k in range(WBN):
        pltpu.sync_copy(acc_sh.at[pl.ds(r0 + k * WB, WB)], rows0)
        pltpu.sync_copy(rows0, out_hbm.at[c, pl.ds(r0 + k * WB, WB)])


def _norm_body(degp_ref, iso_ref, isi_ref):
    dsrc = degp_ref[0]
    ddst = degp_ref[1]
    iso_ref[...] = lax.rsqrt(
        jnp.maximum(jnp.max(dsrc, axis=1, keepdims=True), 1.0))
    isi_ref[...] = lax.rsqrt(
        jnp.maximum(jnp.max(ddst, axis=1, keepdims=True), 1.0))


_norm = pl.pallas_call(
    _norm_body,
    grid=(GRID,),
    in_specs=[pl.BlockSpec((NC, BN, D_H), lambda i: (0, i, 0))],
    out_specs=[
        pl.BlockSpec((BN, 1), lambda i: (i, 0)),
        pl.BlockSpec((BN, 1), lambda i: (i, 0)),
    ],
    out_shape=[
        jax.ShapeDtypeStruct((NP, 1), jnp.float32),
        jax.ShapeDtypeStruct((NP, 1), jnp.float32),
    ],
)


def _mm_scale_body(h_ref, w_ref, iso_ref, o_ref):
    o_ref[...] = jnp.dot(
        h_ref[...], w_ref[...], preferred_element_type=jnp.float32
    ) * iso_ref[...]


_mm1 = pl.pallas_call(
    _mm_scale_body,
    grid=(GRID,),
    in_specs=[
        pl.BlockSpec((BN, D_IN), lambda i: (i, 0)),
        pl.BlockSpec((D_IN, D_H), lambda i: (0, 0)),
        pl.BlockSpec((BN, 1), lambda i: (i, 0)),
    ],
    out_specs=pl.BlockSpec((BN, D_H), lambda i: (i, 0)),
    out_shape=jax.ShapeDtypeStruct((NP, D_H), jnp.float32),
)


def _mid_body(p_ref, isi_ref, b_ref, w_ref, iso_ref, o_ref):
    h = jnp.maximum(
        (p_ref[0] + p_ref[1]) * isi_ref[...] + b_ref[...], 0.0)
    o_ref[...] = jnp.dot(
        h, w_ref[...], preferred_element_type=jnp.float32) * iso_ref[...]


def _make_mid():
    return pl.pallas_call(
        _mid_body,
        grid=(GRID,),
        in_specs=[
            pl.BlockSpec((NC, BN, D_H), lambda i: (0, i, 0)),
            pl.BlockSpec((BN, 1), lambda i: (i, 0)),
            pl.BlockSpec((1, D_H), lambda i: (0, 0)),
            pl.BlockSpec((D_H, D_H), lambda i: (0, 0)),
            pl.BlockSpec((BN, 1), lambda i: (i, 0)),
        ],
        out_specs=pl.BlockSpec((BN, D_H), lambda i: (i, 0)),
        out_shape=jax.ShapeDtypeStruct((NP, D_H), jnp.float32),
    )


def _final_body(p_ref, isi_ref, b_ref, o_ref):
    s = p_ref[0, :, :D_OUT] + p_ref[1, :, :D_OUT]
    o_ref[...] = s * isi_ref[...] + b_ref[...]


_final = pl.pallas_call(
    _final_body,
    grid=(GRID,),
    in_specs=[
        pl.BlockSpec((NC, BN, D_H), lambda i: (0, i, 0)),
        pl.BlockSpec((BN, 1), lambda i: (i, 0)),
        pl.BlockSpec((1, D_OUT), lambda i: (0, 0)),
    ],
    out_specs=pl.BlockSpec((BN, D_OUT), lambda i: (i, 0)),
    out_shape=jax.ShapeDtypeStruct((NP, D_OUT), jnp.float32),
)

_mid2 = _make_mid()
_mid3 = _make_mid()


def kernel(x, edge_index, W1, b1, W2, b2, W3, b3):
    pad_idx = jnp.full((EP - E,), NP - 1, jnp.int32)
    src = jnp.concatenate([edge_index[0], pad_idx])
    dst = jnp.concatenate([edge_index[1], pad_idx])
    x_p = jnp.pad(x, ((0, NP - N), (0, 0)))
    W3p = jnp.pad(W3, ((0, 0), (0, D_H - D_OUT)))
    edge_flat = jnp.concatenate([src, dst])
    degp = _degree_kernel(edge_flat)
    iso, isi = _norm(degp)
    g1 = _mm1(x_p, W1, iso)
    p1 = _agg(g1, src, dst)
    g2 = _mid2(p1, isi, b1.reshape(1, D_H), W2, iso)
    p2 = _agg(g2, src, dst)
    g3 = _mid3(p2, isi, b2.reshape(1, D_H), W3p, iso)
    p3 = _agg(g3, src, dst)
    return _final(p3, isi, b3.reshape(1, D_OUT))[:N]


# speedup vs baseline: 2.1411x; 2.1411x over previous
import functools

import jax
import jax.numpy as jnp
from jax import lax
from jax.experimental import pallas as pl
from jax.experimental.pallas import tpu as pltpu
from jax.experimental.pallas import tpu_sc as plsc

N = 10000
E = 320000
D_IN = 128
D_H = 128
D_OUT = 64

NC = 2
NS = 16
NTILES = NC * NS
NP = 10240
EP = NP * NTILES
CHUNK = 128
EPA = EP // NTILES
NCHA = EPA // CHUNK
EPD = EP // NS
NCHD = EPD // CHUNK
RPT = NP // NS
WB = 128
WBN = RPT // WB

_MESH = plsc.VectorSubcoreMesh(
    core_axis_name="c", subcore_axis_name="s", num_cores=NC, num_subcores=NS
)

BN = 1024
GRID = NP // BN


@functools.partial(
    pl.kernel,
    out_type=jax.ShapeDtypeStruct((NC, NP, D_H), jnp.float32),
    mesh=_MESH,
    scratch_types=[
        pltpu.VMEM((CHUNK,), jnp.int32),
        pltpu.VMEM((CHUNK, D_H), jnp.float32),
        pltpu.VMEM((WB, D_H), jnp.float32),
        pltpu.VMEM_SHARED((NP, D_H), jnp.float32),
    ],
)
def _degree_kernel(ef_hbm, out_hbm, sidx, ones_v, zb, acc_sh):
    c = lax.axis_index("c")
    s = lax.axis_index("s")
    zero = jnp.zeros((16,), jnp.float32)
    one = jnp.ones((16,), jnp.float32)

    @pl.loop(0, WB)
    def _(i):
        for j in range(D_H // 16):
            zb[i, pl.ds(j * 16, 16)] = zero

    @pl.loop(0, CHUNK)
    def _(i):
        for j in range(D_H // 16):
            ones_v[i, pl.ds(j * 16, 16)] = one

    r0 = s * RPT
    for k in range(WBN):
        pltpu.sync_copy(zb, acc_sh.at[pl.ds(r0 + k * WB, WB)])
    plsc.subcore_barrier()

    base = c * EP + s * EPD

    @pl.loop(0, NCHD)
    def _(ch):
        off = base + ch * CHUNK
        pltpu.sync_copy(ef_hbm.at[pl.ds(off, CHUNK)], sidx)
        pltpu.sync_copy(ones_v, acc_sh.at[sidx], add=True)

    plsc.subcore_barrier()
    for k in range(WBN):
        pltpu.sync_copy(acc_sh.at[pl.ds(r0 + k * WB, WB)], zb)
        pltpu.sync_copy(zb, out_hbm.at[c, pl.ds(r0 + k * WB, WB)])


@functools.partial(
    pl.kernel,
    out_type=jax.ShapeDtypeStruct((NC, NP, D_H), jnp.float32),
    mesh=_MESH,
    scratch_types=[
        pltpu.VMEM((CHUNK,), jnp.int32),
        pltpu.VMEM((CHUNK,), jnp.int32),
        pltpu.VMEM((CHUNK,), jnp.int32),
        pltpu.VMEM((CHUNK,), jnp.int32),
        pltpu.VMEM((CHUNK, D_H), jnp.float32),
        pltpu.VMEM((CHUNK, D_H), jnp.float32),
        pltpu.VMEM_SHARED((NP, D_H), jnp.float32),
        pltpu.SemaphoreType.DMA,
        pltpu.SemaphoreType.DMA,
    ],
)
def _agg(g_hbm, src_hbm, dst_hbm, out_hbm, sidx0, didx0, sidx1, didx1,
         rows0, rows1, acc_sh, sem0, sem1):
    c = lax.axis_index("c")
    s = lax.axis_index("s")
    tid = s * NC + c
    zero = jnp.zeros((16,), jnp.float32)

    @pl.loop(0, WB)
    def _(i):
        for j in range(D_H // 16):
            rows0[i, pl.ds(j * 16, 16)] = zero

    r0 = s * RPT
    for k in range(WBN):
        pltpu.sync_copy(rows0, acc_sh.at[pl.ds(r0 + k * WB, WB)])
    plsc.subcore_barrier()

    base = tid * EPA

    pltpu.sync_copy(src_hbm.at[pl.ds(base, CHUNK)], sidx0)
    pltpu.sync_copy(dst_hbm.at[pl.ds(base, CHUNK)], didx0)
    pltpu.async_copy(g_hbm.at[sidx0], rows0, sem0)

    nhalf = NCHA // 2

    @pl.loop(0, nhalf)
    def _(i):
        ch = i * 2
        off1 = base + (ch + 1) * CHUNK
        pltpu.sync_copy(src_hbm.at[pl.ds(off1, CHUNK)], sidx1)
        pltpu.sync_copy(dst_hbm.at[pl.ds(off1, CHUNK)], didx1)
        pltpu.make_async_copy(g_hbm.at[sidx0], rows0, sem0).wait()
        pltpu.async_copy(g_hbm.at[sidx1], rows1, sem1)
        pltpu.sync_copy(rows0, acc_sh.at[didx0], add=True)

        @pl.when(i < nhalf - 1)
        def _():
            off2 = base + (ch + 2) * CHUNK
            pltpu.sync_copy(src_hbm.at[pl.ds(off2, CHUNK)], sidx0)
            pltpu.sync_copy(dst_hbm.at[pl.ds(off2, CHUNK)], didx0)

        pltpu.make_async_copy(g_hbm.at[sidx1], rows1, sem1).wait()

        @pl.when(i < nhalf - 1)
        def _():
            pltpu.async_copy(g_hbm.at[sidx0], rows0, sem0)

        pltpu.sync_copy(rows1, acc_sh.at[didx1], add=True)

    plsc.subcore_barrier()
    for k in range(WBN):
        pltpu.sync_copy(acc_sh.at[pl.ds(r0 + k * WB, WB)], rows0)
        pltpu.sync_copy(rows0, out_hbm.at[c, pl.ds(r0 + k * WB, WB)])


def _norm_body(degp_ref, iso_ref, isi_ref):
    dsrc = degp_ref[0]
    ddst = degp_ref[1]
    iso_ref[...] = lax.rsqrt(
        jnp.maximum(jnp.max(dsrc, axis=1, keepdims=True), 1.0))
    isi_ref[...] = lax.rsqrt(
        jnp.maximum(jnp.max(ddst, axis=1, keepdims=True), 1.0))


_norm = pl.pallas_call(
    _norm_body,
    grid=(GRID,),
    in_specs=[pl.BlockSpec((NC, BN, D_H), lambda i: (0, i, 0))],
    out_specs=[
        pl.BlockSpec((BN, 1), lambda i: (i, 0)),
        pl.BlockSpec((BN, 1), lambda i: (i, 0)),
    ],
    out_shape=[
        jax.ShapeDtypeStruct((NP, 1), jnp.float32),
        jax.ShapeDtypeStruct((NP, 1), jnp.float32),
    ],
)


def _mm_scale_body(h_ref, w_ref, iso_ref, o_ref):
    o_ref[...] = jnp.dot(
        h_ref[...], w_ref[...], preferred_element_type=jnp.float32
    ) * iso_ref[...]


_mm1 = pl.pallas_call(
    _mm_scale_body,
    grid=(GRID,),
    in_specs=[
        pl.BlockSpec((BN, D_IN), lambda i: (i, 0)),
        pl.BlockSpec((D_IN, D_H), lambda i: (0, 0)),
        pl.BlockSpec((BN, 1), lambda i: (i, 0)),
    ],
    out_specs=pl.BlockSpec((BN, D_H), lambda i: (i, 0)),
    out_shape=jax.ShapeDtypeStruct((NP, D_H), jnp.float32),
)


def _mid_body(p_ref, isi_ref, b_ref, w_ref, iso_ref, o_ref):
    h = jnp.maximum(
        (p_ref[0] + p_ref[1]) * isi_ref[...] + b_ref[...], 0.0)
    o_ref[...] = jnp.dot(
        h, w_ref[...], preferred_element_type=jnp.float32) * iso_ref[...]


def _make_mid():
    return pl.pallas_call(
        _mid_body,
        grid=(GRID,),
        in_specs=[
            pl.BlockSpec((NC, BN, D_H), lambda i: (0, i, 0)),
            pl.BlockSpec((BN, 1), lambda i: (i, 0)),
            pl.BlockSpec((1, D_H), lambda i: (0, 0)),
            pl.BlockSpec((D_H, D_H), lambda i: (0, 0)),
            pl.BlockSpec((BN, 1), lambda i: (i, 0)),
        ],
        out_specs=pl.BlockSpec((BN, D_H), lambda i: (i, 0)),
        out_shape=jax.ShapeDtypeStruct((NP, D_H), jnp.float32),
    )


def _final_body(p_ref, isi_ref, b_ref, o_ref):
    s = p_ref[0, :, :D_OUT] + p_ref[1, :, :D_OUT]
    o_ref[...] = s * isi_ref[...] + b_ref[...]


_final = pl.pallas_call(
    _final_body,
    grid=(GRID,),
    in_specs=[
        pl.BlockSpec((NC, BN, D_H), lambda i: (0, i, 0)),
        pl.BlockSpec((BN, 1), lambda i: (i, 0)),
        pl.BlockSpec((1, D_OUT), lambda i: (0, 0)),
    ],
    out_specs=pl.BlockSpec((BN, D_OUT), lambda i: (i, 0)),
    out_shape=jax.ShapeDtypeStruct((NP, D_OUT), jnp.float32),
)

_mid2 = _make_mid()
_mid3 = _make_mid()


def kernel(x, edge_index, W1, b1, W2, b2, W3, b3):
    pad_idx = N + jnp.arange(EP - E, dtype=jnp.int32) % (NP - N)
    src = jnp.concatenate([edge_index[0], pad_idx])
    dst = jnp.concatenate([edge_index[1], pad_idx])
    x_p = jnp.pad(x, ((0, NP - N), (0, 0)))
    W3p = jnp.pad(W3, ((0, 0), (0, D_H - D_OUT)))
    edge_flat = jnp.concatenate([src, dst])
    degp = _degree_kernel(edge_flat)
    iso, isi = _norm(degp)
    g1 = _mm1(x_p, W1, iso)
    p1 = _agg(g1, src, dst)
    g2 = _mid2(p1, isi, b1.reshape(1, D_H), W2, iso)
    p2 = _agg(g2, src, dst)
    g3 = _mid3(p2, isi, b2.reshape(1, D_H), W3p, iso)
    p3 = _agg(g3, src, dst)
    return _final(p3, isi, b3.reshape(1, D_OUT))[:N]
